# Initial kernel scaffold; baseline (speedup 1.0000x reference)
import jax
import jax.numpy as jnp
from jax.experimental import pallas as pl


def kernel(x, table):
    raise NotImplementedError("write your pallas kernel here")


# speedup vs baseline: 1.0953x; 1.0953x over previous
import functools

import jax
import jax.numpy as jnp
from jax import lax
from jax.experimental import pallas as pl
from jax.experimental.pallas import tpu as pltpu
from jax.experimental.pallas import tpu_sc as plsc

_D = 32
_B = 16384 * 50

_info = plsc.get_sparse_core_info()
_NC, _NS = _info.num_cores, _info.num_subcores
_NW = _NC * _NS
_B_PER_W = _B // _NW
_CHUNK = 1024
_NITER = _B_PER_W // _CHUNK


@functools.partial(
    pl.kernel,
    out_type=jax.ShapeDtypeStruct((_B, _D), jnp.float32),
    mesh=plsc.VectorSubcoreMesh(core_axis_name="c", subcore_axis_name="s"),
    scratch_types=[
        pltpu.VMEM((_CHUNK,), jnp.int32),
        pltpu.VMEM((_CHUNK, _D), jnp.float32),
        pltpu.SemaphoreType.DMA,
    ],
    compiler_params=pltpu.CompilerParams(use_tc_tiling_on_sc=False),
)
def _emb_gather(idx_hbm, table_hbm, out_hbm, idx_v, rows_v, sem):
    wid = lax.axis_index("s") * _NC + lax.axis_index("c")
    base = wid * _B_PER_W

    def body(i, carry):
        off = base + i * _CHUNK
        pltpu.sync_copy(idx_hbm.at[pl.ds(off, _CHUNK)], idx_v)
        pltpu.async_copy(table_hbm.at[idx_v], rows_v, sem).wait()
        pltpu.sync_copy(rows_v, out_hbm.at[pl.ds(off, _CHUNK)])
        return carry

    lax.fori_loop(0, _NITER, body, 0)


def kernel(x, table):
    flat = x.reshape(-1).astype(jnp.int32)
    out = _emb_gather(flat, table)
    return out.reshape(x.shape + (_D,))


# speedup vs baseline: 1.1128x; 1.0160x over previous
import functools

import jax
import jax.numpy as jnp
from jax import lax
from jax.experimental import pallas as pl
from jax.experimental.pallas import tpu as pltpu
from jax.experimental.pallas import tpu_sc as plsc

_D = 32
_B = 16384 * 50

_info = plsc.get_sparse_core_info()
_NC, _NS = _info.num_cores, _info.num_subcores
_NW = _NC * _NS
_B_PER_W = _B // _NW
_CHUNK = 800
_NB = 4
_L = 1
_N = _B_PER_W // _CHUNK
_T = _N // _NB


@functools.partial(
    pl.kernel,
    out_type=jax.ShapeDtypeStruct((_B, _D), jnp.float32),
    mesh=plsc.VectorSubcoreMesh(core_axis_name="c", subcore_axis_name="s"),
    scratch_types=[
        pltpu.VMEM((_B_PER_W,), jnp.int32),
        pltpu.VMEM((_NB, _CHUNK, _D), jnp.float32),
    ]
    + [pltpu.SemaphoreType.DMA] * (2 * _NB),
    compiler_params=pltpu.CompilerParams(use_tc_tiling_on_sc=False),
)
def _emb_gather(idx_hbm, table_hbm, out_hbm, idx_all, rows, *sems):
    g_sems, s_sems = sems[:_NB], sems[_NB:]
    wid = lax.axis_index("s") * _NC + lax.axis_index("c")
    base = wid * _B_PER_W

    pltpu.sync_copy(idx_hbm.at[pl.ds(base, _B_PER_W)], idx_all)

    def start_gather(g, b):
        idx_sl = idx_all.at[pl.ds(g * _CHUNK, _CHUNK)]
        pltpu.async_copy(table_hbm.at[idx_sl], rows.at[b], g_sems[b])

    def wait_gather(b):
        idx_sl = idx_all.at[pl.ds(0, _CHUNK)]
        pltpu.make_async_copy(table_hbm.at[idx_sl], rows.at[b], g_sems[b]).wait()

    def start_store(g, b):
        dst = out_hbm.at[pl.ds(base + g * _CHUNK, _CHUNK)]
        pltpu.async_copy(rows.at[b], dst, s_sems[b])

    def wait_store(b):
        dst = out_hbm.at[pl.ds(base, _CHUNK)]
        pltpu.make_async_copy(rows.at[b], dst, s_sems[b]).wait()

    for b in range(_NB):
        start_gather(b, b)
        if b >= _L:
            qb = (b - _L) % _NB
            wait_gather(qb)
            start_store(b - _L, qb)

    def group(t, carry):
        g0 = t * _NB
        for b in range(_NB):
            g = g0 + b
            wait_store(b)
            start_gather(g, b)
            qb = (b - _L) % _NB
            wait_gather(qb)
            start_store(g - _L, qb)
        return carry

    lax.fori_loop(1, _T, group, 0)

    for i in range(_L):
        g = _N - _L + i
        b = g % _NB
        wait_gather(b)
        start_store(g, b)
    for b in range(_NB):
        wait_store(b)


def kernel(x, table):
    flat = x.reshape(-1).astype(jnp.int32)
    out = _emb_gather(flat, table)
    return out.reshape(x.shape + (_D,))


# speedup vs baseline: 1.6997x; 1.5273x over previous
import functools

import jax
import jax.numpy as jnp
from jax import lax
from jax.experimental import pallas as pl
from jax.experimental.pallas import tpu as pltpu
from jax.experimental.pallas import tpu_sc as plsc

_D = 32
_R = 16384
_C = 50

_info = plsc.get_sparse_core_info()
_NC, _NS = _info.num_cores, _info.num_subcores
_NW = _NC * _NS
_R_PER_W = _R // _NW
_CR = 16
_CHUNK = _CR * _C
_N = _R_PER_W // _CR


@functools.partial(
    pl.kernel,
    out_type=jax.ShapeDtypeStruct((_R, _C, _D), jnp.float32),
    mesh=plsc.VectorSubcoreMesh(core_axis_name="c", subcore_axis_name="s"),
    scratch_types=[
        pltpu.VMEM((_CR, _C), jnp.int32),
        pltpu.VMEM((_CHUNK,), jnp.int32),
        pltpu.VMEM((_CHUNK, _D), jnp.float32),
        pltpu.SemaphoreType.DMA,
    ],
    compiler_params=pltpu.CompilerParams(use_tc_tiling_on_sc=False),
)
def _emb_gather(x_hbm, table_hbm, out_hbm, cidx, fidx, rows, sem):
    wid = lax.axis_index("s") * _NC + lax.axis_index("c")
    row0 = wid * _R_PER_W

    def body(g, carry):
        r0 = row0 + g * _CR
        pltpu.sync_copy(x_hbm.at[pl.ds(r0, _CR), :], cidx)
        for r in range(_CR):
            for k in (0, 16, 32, 34):
                fidx[pl.ds(r * _C + k, 16)] = cidx[r, pl.ds(k, 16)]
        pltpu.async_copy(table_hbm.at[fidx], rows, sem).wait()
        for r in range(_CR):
            pltpu.sync_copy(rows.at[pl.ds(r * _C, _C), :], out_hbm.at[r0 + r])
        return carry

    lax.fori_loop(0, _N, body, 0)


def kernel(x, table):
    return _emb_gather(x.astype(jnp.int32), table)
